# Initial kernel scaffold; baseline (speedup 1.0000x reference)
import jax
import jax.numpy as jnp
from jax.experimental import pallas as pl


def kernel(x, table, W_ih_f, W_hh_f, b_ih_f, b_hh_f, W_ih_b, W_hh_b, b_ih_b, b_hh_b):
    raise NotImplementedError("write your pallas kernel here")


# speedup vs baseline: 2.3635x; 2.3635x over previous
import functools

import jax
import jax.numpy as jnp
from jax import lax
from jax.experimental import pallas as pl
from jax.experimental.pallas import tpu as pltpu
from jax.experimental.pallas import tpu_sc as plsc

VOCAB = 1000000
EMB = 64
HID = 64
B = 4096
T = 200


_NC, _NS = 2, 16
_NW = _NC * _NS
_N_ROWS = B * T
_PER_W = _N_ROWS // _NW
_CH = 128
_NCHUNK = _PER_W // _CH


def _sc_gather(table, idx_flat):
    mesh = plsc.VectorSubcoreMesh(core_axis_name="c", subcore_axis_name="s")

    @functools.partial(
        pl.kernel,
        mesh=mesh,
        compiler_params=pltpu.CompilerParams(use_tc_tiling_on_sc=False),
        out_type=jax.ShapeDtypeStruct((_N_ROWS, EMB), jnp.float32),
        scratch_types=[
            pltpu.VMEM((_NCHUNK, _CH), jnp.int32),
            pltpu.VMEM((_CH, EMB), jnp.float32),
            pltpu.SemaphoreType.DMA,
        ],
    )
    def k(table_hbm, idx_hbm, out_hbm, idx_v, rows_v, sem):
        wid = lax.axis_index("s") * _NC + lax.axis_index("c")
        base = wid * _PER_W
        pltpu.sync_copy(idx_hbm.at[pl.ds(wid * _NCHUNK, _NCHUNK)], idx_v)

        def chunk(j, _):
            pltpu.async_copy(table_hbm.at[idx_v.at[j]], rows_v, sem).wait()
            pltpu.sync_copy(rows_v, out_hbm.at[pl.ds(base + j * _CH, _CH)])
            return _

        lax.fori_loop(0, _NCHUNK, chunk, 0)

    return k(table, idx_flat)


def _gru_body(emb_ref, wih_ref, whh_ref, bih_ref, bhh_ref, out_ref, h_ref):
    t = pl.program_id(1)

    @pl.when(t == 0)
    def _():
        h_ref[...] = jnp.zeros((B, HID), jnp.float32)

    x_t = emb_ref[0]
    h = h_ref[...]
    wih = wih_ref[0]
    whh = whh_ref[0]
    gi = lax.dot_general(x_t, wih, (((1,), (1,)), ((), ())),
                         preferred_element_type=jnp.float32) + bih_ref[0]
    gh = lax.dot_general(h, whh, (((1,), (1,)), ((), ())),
                         preferred_element_type=jnp.float32) + bhh_ref[0]
    r = jax.nn.sigmoid(gi[:, :HID] + gh[:, :HID])
    z = jax.nn.sigmoid(gi[:, HID:2 * HID] + gh[:, HID:2 * HID])
    n = jnp.tanh(gi[:, 2 * HID:] + r * gh[:, 2 * HID:])
    h_new = (1.0 - z) * n + z * h
    h_ref[...] = h_new
    out_ref[0, 0] = h_new


def _gru_bidir(emb_tbe, wih_s, whh_s, bih_s, bhh_s):
    def t_idx(d, t):
        return jnp.where(d == 0, t, T - 1 - t)

    return pl.pallas_call(
        _gru_body,
        grid=(2, T),
        in_specs=[
            pl.BlockSpec((1, B, EMB), lambda d, t: (t_idx(d, t), 0, 0)),
            pl.BlockSpec((1, 3 * HID, EMB), lambda d, t: (d, 0, 0)),
            pl.BlockSpec((1, 3 * HID, HID), lambda d, t: (d, 0, 0)),
            pl.BlockSpec((1, 1, 3 * HID), lambda d, t: (d, 0, 0)),
            pl.BlockSpec((1, 1, 3 * HID), lambda d, t: (d, 0, 0)),
        ],
        out_specs=pl.BlockSpec((1, 1, B, HID),
                               lambda d, t: (d, t_idx(d, t), 0, 0)),
        out_shape=jax.ShapeDtypeStruct((2, T, B, HID), jnp.float32),
        scratch_shapes=[pltpu.VMEM((B, HID), jnp.float32)],
    )(emb_tbe, wih_s, whh_s, bih_s, bhh_s)


def kernel(x, table, W_ih_f, W_hh_f, b_ih_f, b_hh_f,
           W_ih_b, W_hh_b, b_ih_b, b_hh_b):
    idx_2d = x.T.reshape(_NW * _NCHUNK, _CH).astype(jnp.int32)
    emb = _sc_gather(table, idx_2d).reshape(T, B, EMB)
    wih_s = jnp.stack([W_ih_f, W_ih_b])
    whh_s = jnp.stack([W_hh_f, W_hh_b])
    bih_s = jnp.stack([b_ih_f, b_ih_b])[:, None, :]
    bhh_s = jnp.stack([b_hh_f, b_hh_b])[:, None, :]
    out = _gru_bidir(emb, wih_s, whh_s, bih_s, bhh_s)
    res = jnp.concatenate([out[0], out[1]], axis=-1)
    return jnp.transpose(res, (1, 0, 2))


# speedup vs baseline: 3.4835x; 1.4739x over previous
import functools

import jax
import jax.numpy as jnp
from jax import lax
from jax.experimental import pallas as pl
from jax.experimental.pallas import tpu as pltpu
from jax.experimental.pallas import tpu_sc as plsc

VOCAB = 1000000
EMB = 64
HID = 64
B = 4096
T = 200


_NC, _NS = 2, 16
_NW = _NC * _NS
_N_ROWS = B * T
_PER_W = _N_ROWS // _NW
_CH = 128
_NCHUNK = _PER_W // _CH


def _sc_gather(table, idx_2d):
    mesh = plsc.VectorSubcoreMesh(core_axis_name="c", subcore_axis_name="s")

    @functools.partial(
        pl.kernel,
        mesh=mesh,
        compiler_params=pltpu.CompilerParams(use_tc_tiling_on_sc=False),
        out_type=jax.ShapeDtypeStruct((_N_ROWS, EMB), jnp.float32),
        scratch_types=[
            pltpu.VMEM((_NCHUNK, _CH), jnp.int32),
            pltpu.VMEM((2, _CH, EMB), jnp.float32),
            pltpu.SemaphoreType.DMA,
            pltpu.SemaphoreType.DMA,
        ],
    )
    def k(table_hbm, idx_hbm, out_hbm, idx_v, rows_v, sem0, sem1):
        wid = lax.axis_index("s") * _NC + lax.axis_index("c")
        base = wid * _PER_W
        pltpu.sync_copy(idx_hbm.at[pl.ds(wid * _NCHUNK, _NCHUNK)], idx_v)
        sems = (sem0, sem1)

        def start(j, slot):
            pltpu.async_copy(table_hbm.at[idx_v.at[j]], rows_v.at[slot],
                             sems[slot])

        start(0, 0)
        start(1, 1)

        def pair(i, carry):
            for b in range(2):
                j = 2 * i + b
                pltpu.make_async_copy(table_hbm.at[idx_v.at[j]],
                                      rows_v.at[b], sems[b]).wait()
                pltpu.sync_copy(rows_v.at[b],
                                out_hbm.at[pl.ds(base + j * _CH, _CH)])

                @pl.when(j + 2 < _NCHUNK)
                def _start_next(b=b, j=j):
                    start(j + 2, b)
            return carry

        lax.fori_loop(0, _NCHUNK // 2, pair, 0)

    return k(table, idx_2d)


def _gru_math(x_t, h, wih, whh, bih, bhh):
    gi = lax.dot_general(x_t, wih, (((1,), (1,)), ((), ())),
                         preferred_element_type=jnp.float32) + bih
    gh = lax.dot_general(h, whh, (((1,), (1,)), ((), ())),
                         preferred_element_type=jnp.float32) + bhh
    r = jax.nn.sigmoid(gi[:, :HID] + gh[:, :HID])
    z = jax.nn.sigmoid(gi[:, HID:2 * HID] + gh[:, HID:2 * HID])
    n = jnp.tanh(gi[:, 2 * HID:] + r * gh[:, 2 * HID:])
    return (1.0 - z) * n + z * h


def _fwd_body(emb_ref, wih_ref, whh_ref, bih_ref, bhh_ref, out_ref, h_ref):
    t = pl.program_id(0)

    @pl.when(t == 0)
    def _():
        h_ref[...] = jnp.zeros((B, HID), jnp.float32)

    h_new = _gru_math(emb_ref[0], h_ref[...], wih_ref[...], whh_ref[...],
                      bih_ref[...], bhh_ref[...])
    h_ref[...] = h_new
    out_ref[0] = h_new


def _bwd_body(emb_ref, fwd_ref, wih_ref, whh_ref, bih_ref, bhh_ref,
              out_ref, h_ref):
    t = pl.program_id(0)

    @pl.when(t == 0)
    def _():
        h_ref[...] = jnp.zeros((B, HID), jnp.float32)

    h_new = _gru_math(emb_ref[0], h_ref[...], wih_ref[...], whh_ref[...],
                      bih_ref[...], bhh_ref[...])
    h_ref[...] = h_new
    j = (T - 1 - t) % 8
    out_ref[:, j, :] = jnp.concatenate([fwd_ref[0], h_new], axis=-1)


def _w_specs():
    return [
        pl.BlockSpec((3 * HID, EMB), lambda t: (0, 0)),
        pl.BlockSpec((3 * HID, HID), lambda t: (0, 0)),
        pl.BlockSpec((1, 3 * HID), lambda t: (0, 0)),
        pl.BlockSpec((1, 3 * HID), lambda t: (0, 0)),
    ]


def _gru_forward(emb_tbe, wih, whh, bih, bhh):
    return pl.pallas_call(
        _fwd_body,
        grid=(T,),
        in_specs=[pl.BlockSpec((1, B, EMB), lambda t: (t, 0, 0))] + _w_specs(),
        out_specs=pl.BlockSpec((1, B, HID), lambda t: (t, 0, 0)),
        out_shape=jax.ShapeDtypeStruct((T, B, HID), jnp.float32),
        scratch_shapes=[pltpu.VMEM((B, HID), jnp.float32)],
    )(emb_tbe, wih, whh, bih, bhh)


def _gru_backward_combine(emb_tbe, out_f, wih, whh, bih, bhh):
    rev = lambda t: (T - 1 - t, 0, 0)
    return pl.pallas_call(
        _bwd_body,
        grid=(T,),
        in_specs=[pl.BlockSpec((1, B, EMB), rev),
                  pl.BlockSpec((1, B, HID), rev)] + _w_specs(),
        out_specs=pl.BlockSpec((B, 8, 2 * HID),
                               lambda t: (0, (T - 1 - t) // 8, 0)),
        out_shape=jax.ShapeDtypeStruct((B, T, 2 * HID), jnp.float32),
        scratch_shapes=[pltpu.VMEM((B, HID), jnp.float32)],
    )(emb_tbe, out_f, wih, whh, bih, bhh)


def kernel(x, table, W_ih_f, W_hh_f, b_ih_f, b_hh_f,
           W_ih_b, W_hh_b, b_ih_b, b_hh_b):
    idx_2d = x.T.reshape(_NW * _NCHUNK, _CH).astype(jnp.int32)
    emb = _sc_gather(table, idx_2d).reshape(T, B, EMB)
    out_f = _gru_forward(emb, W_ih_f, W_hh_f, b_ih_f[None, :], b_hh_f[None, :])
    return _gru_backward_combine(emb, out_f, W_ih_b, W_hh_b,
                                 b_ih_b[None, :], b_hh_b[None, :])
